# baseline (speedup 1.0000x reference)
import functools

import jax
import jax.numpy as jnp
from jax import lax
from jax.experimental import pallas as pl
from jax.experimental.pallas import tpu as pltpu
from jax.experimental.pallas import tpu_sc as plsc

_VOCAB = 100000
_DIM = 16
_BATCH = 1024
_TV = 1024


@functools.cache
def _sc_gather_kernel():
    info = plsc.get_sparse_core_info()
    nc, ns = info.num_cores, info.num_subcores
    nw = nc * ns
    b_per_w = _BATCH // nw
    mesh = plsc.VectorSubcoreMesh(core_axis_name="c", subcore_axis_name="s")

    @functools.partial(
        pl.kernel,
        mesh=mesh,
        out_type=jax.ShapeDtypeStruct((_DIM + 1, _BATCH), jnp.float32),
        scratch_types=[
            pltpu.VMEM((_DIM, b_per_w), jnp.int32),
            pltpu.VMEM((_DIM + 1, b_per_w), jnp.float32),
            pltpu.SemaphoreType.DMA,
        ],
        compiler_params=pltpu.CompilerParams(use_tc_tiling_on_sc=False),
    )
    def gather(flat_hbm, idx2_hbm, out_hbm, idx_v, rows_v, sem):
        wid = lax.axis_index("s") * nc + lax.axis_index("c")
        base = wid * b_per_w
        pltpu.sync_copy(idx2_hbm.at[:, pl.ds(base, b_per_w)], idx_v)
        copies = [
            pltpu.async_copy(flat_hbm.at[idx_v.at[k]], rows_v.at[k], sem)
            for k in range(_DIM)
        ]
        for g in range(b_per_w // 16):
            rows_v[_DIM, pl.ds(g * 16, 16)] = jnp.ones((16,), jnp.float32)
        for c in copies:
            c.wait()
        pltpu.sync_copy(rows_v, out_hbm.at[:, pl.ds(base, b_per_w)])

    return gather


def _matmul_t_body(w_ref, b_ref, e_ref, out_ref):
    lhs = jnp.concatenate([w_ref[...], b_ref[...]], axis=0)
    out_ref[...] = jax.lax.dot_general(
        lhs,
        e_ref[...],
        dimension_numbers=(((0,), (0,)), ((), ())),
        preferred_element_type=jnp.float32,
    )


def kernel(center_word_idx, emb_table, out_weight, out_bias):
    idx = center_word_idx.astype(jnp.int32)
    idx2 = idx[None, :] + (jnp.arange(_DIM, dtype=jnp.int32) * _VOCAB)[:, None]
    flat_table = emb_table.T.reshape(-1)
    emb_aug = _sc_gather_kernel()(flat_table, idx2)
    w_t = out_weight.T
    bias2d = out_bias.reshape(1, _VOCAB)
    out_t = pl.pallas_call(
        _matmul_t_body,
        grid=(pl.cdiv(_VOCAB, _TV),),
        in_specs=[
            pl.BlockSpec((_DIM, _TV), lambda i: (0, i)),
            pl.BlockSpec((1, _TV), lambda i: (0, i)),
            pl.BlockSpec((_DIM + 1, _BATCH), lambda i: (0, 0)),
        ],
        out_specs=pl.BlockSpec((_TV, _BATCH), lambda i: (i, 0)),
        out_shape=jax.ShapeDtypeStruct((_VOCAB, _BATCH), jnp.float32),
    )(w_t, bias2d, emb_aug)
    return out_t.T


# speedup vs baseline: 1.0938x; 1.0938x over previous
import functools

import jax
import jax.numpy as jnp
from jax import lax
from jax.experimental import pallas as pl
from jax.experimental.pallas import tpu as pltpu
from jax.experimental.pallas import tpu_sc as plsc

_VOCAB = 100000
_DIM = 16
_BATCH = 1024
_TV = 2048


@functools.cache
def _sc_gather_kernel():
    info = plsc.get_sparse_core_info()
    nc, ns = info.num_cores, info.num_subcores
    nw = nc * ns
    b_per_w = _BATCH // nw
    mesh = plsc.VectorSubcoreMesh(core_axis_name="c", subcore_axis_name="s")

    @functools.partial(
        pl.kernel,
        mesh=mesh,
        out_type=jax.ShapeDtypeStruct((_DIM + 1, _BATCH), jnp.float32),
        scratch_types=[
            pltpu.VMEM((_DIM, b_per_w), jnp.int32),
            pltpu.VMEM((_DIM + 1, b_per_w), jnp.float32),
            pltpu.SemaphoreType.DMA,
        ],
        compiler_params=pltpu.CompilerParams(use_tc_tiling_on_sc=False),
    )
    def gather(flat_hbm, idx2_hbm, out_hbm, idx_v, rows_v, sem):
        wid = lax.axis_index("s") * nc + lax.axis_index("c")
        base = wid * b_per_w
        pltpu.sync_copy(idx2_hbm.at[:, pl.ds(base, b_per_w)], idx_v)
        copies = [
            pltpu.async_copy(flat_hbm.at[idx_v.at[k]], rows_v.at[k], sem)
            for k in range(_DIM)
        ]
        for g in range(b_per_w // 16):
            rows_v[_DIM, pl.ds(g * 16, 16)] = jnp.ones((16,), jnp.float32)
        for c in copies:
            c.wait()
        pltpu.sync_copy(rows_v, out_hbm.at[:, pl.ds(base, b_per_w)])

    return gather


def _matmul_t_body(w_ref, b_ref, e_ref, out_ref):
    lhs = jnp.concatenate([w_ref[...], b_ref[...]], axis=0)
    out_ref[...] = jax.lax.dot_general(
        lhs,
        e_ref[...],
        dimension_numbers=(((0,), (0,)), ((), ())),
        preferred_element_type=jnp.float32,
    )


def kernel(center_word_idx, emb_table, out_weight, out_bias):
    idx = center_word_idx.astype(jnp.int32)
    idx2 = idx[None, :] + (jnp.arange(_DIM, dtype=jnp.int32) * _VOCAB)[:, None]
    flat_table = emb_table.T.reshape(-1)
    emb_aug = _sc_gather_kernel()(flat_table, idx2)
    w_t = out_weight.T
    bias2d = out_bias.reshape(1, _VOCAB)
    out_t = pl.pallas_call(
        _matmul_t_body,
        grid=(pl.cdiv(_VOCAB, _TV),),
        in_specs=[
            pl.BlockSpec((_DIM, _TV), lambda i: (0, i)),
            pl.BlockSpec((1, _TV), lambda i: (0, i)),
            pl.BlockSpec((_DIM + 1, _BATCH), lambda i: (0, 0)),
        ],
        out_specs=pl.BlockSpec((_TV, _BATCH), lambda i: (i, 0)),
        out_shape=jax.ShapeDtypeStruct((_VOCAB, _BATCH), jnp.float32),
    )(w_t, bias2d, emb_aug)
    return out_t.T


# speedup vs baseline: 1.1020x; 1.0075x over previous
import functools

import jax
import jax.numpy as jnp
from jax import lax
from jax.experimental import pallas as pl
from jax.experimental.pallas import tpu as pltpu
from jax.experimental.pallas import tpu_sc as plsc

_VOCAB = 100000
_DIM = 16
_BATCH = 1024
_TV = 2048


@functools.cache
def _sc_gather_kernel():
    info = plsc.get_sparse_core_info()
    nc, ns = info.num_cores, info.num_subcores
    nw = nc * ns
    b_per_w = _BATCH // nw
    mesh = plsc.VectorSubcoreMesh(core_axis_name="c", subcore_axis_name="s")

    @functools.partial(
        pl.kernel,
        mesh=mesh,
        out_type=jax.ShapeDtypeStruct((_DIM + 1, _BATCH), jnp.float32),
        scratch_types=[
            pltpu.VMEM((b_per_w,), jnp.int32),
            pltpu.VMEM((_DIM + 1, b_per_w), jnp.float32),
            pltpu.SemaphoreType.DMA,
        ],
        compiler_params=pltpu.CompilerParams(use_tc_tiling_on_sc=False),
    )
    def gather(flat_hbm, idx_hbm, out_hbm, idx_v, rows_v, sem):
        wid = lax.axis_index("s") * nc + lax.axis_index("c")
        base = wid * b_per_w
        pltpu.sync_copy(idx_hbm.at[pl.ds(base, b_per_w)], idx_v)
        copies = []
        for g in range(b_per_w // 16):
            v = idx_v[pl.ds(g * 16, 16)]
            for k in range(_DIM):
                copies.append(
                    pltpu.async_copy(
                        flat_hbm.at[v + k * _VOCAB],
                        rows_v.at[k, pl.ds(g * 16, 16)],
                        sem,
                    )
                )
        for g in range(b_per_w // 16):
            rows_v[_DIM, pl.ds(g * 16, 16)] = jnp.ones((16,), jnp.float32)
        for c in copies:
            c.wait()
        pltpu.sync_copy(rows_v, out_hbm.at[:, pl.ds(base, b_per_w)])

    return gather


def _matmul_t_body(w_ref, b_ref, e_ref, out_ref):
    lhs = jnp.concatenate([w_ref[...], b_ref[...]], axis=0)
    out_ref[...] = jax.lax.dot_general(
        lhs,
        e_ref[...],
        dimension_numbers=(((0,), (0,)), ((), ())),
        preferred_element_type=jnp.float32,
    )


def kernel(center_word_idx, emb_table, out_weight, out_bias):
    idx = center_word_idx.astype(jnp.int32)
    flat_table = emb_table.T.reshape(-1)
    emb_aug = _sc_gather_kernel()(flat_table, idx)
    w_t = out_weight.T
    bias2d = out_bias.reshape(1, _VOCAB)
    out_t = pl.pallas_call(
        _matmul_t_body,
        grid=(pl.cdiv(_VOCAB, _TV),),
        in_specs=[
            pl.BlockSpec((_DIM, _TV), lambda i: (0, i)),
            pl.BlockSpec((1, _TV), lambda i: (0, i)),
            pl.BlockSpec((_DIM + 1, _BATCH), lambda i: (0, 0)),
        ],
        out_specs=pl.BlockSpec((_TV, _BATCH), lambda i: (i, 0)),
        out_shape=jax.ShapeDtypeStruct((_VOCAB, _BATCH), jnp.float32),
    )(w_t, bias2d, emb_aug)
    return out_t.T
